# baseline (speedup 1.0000x reference)
import functools

import jax
import jax.numpy as jnp
from jax import lax
from jax.experimental import pallas as pl
from jax.experimental.pallas import tpu as pltpu
from jax.experimental.pallas import tpu_sc as plsc

N_NODES = 10000
N_EDGES = 10000
N_INC = 320000
CH = 128

NC = 2
NS = 16
NW = NC * NS
LANES = 16

INC_PER_W = N_INC // NW
INC_PER_T = N_INC // NS
CHUNK = 32
N_CHUNKS = 626
INC_PAD = N_CHUNKS * CHUNK
OWN = 5120
ACC_N = 2 * OWN
PAD_ROW = ACC_N - 1
ACC_H = 5248
DEAD = ACC_H - 1
ZPT = ACC_H // NS
DPT = OWN // NS

_mesh = plsc.VectorSubcoreMesh(core_axis_name="c", subcore_axis_name="s")
_sc_params = pltpu.CompilerParams(needs_layout_passes=False)


@functools.partial(
    pl.kernel,
    mesh=_mesh,
    out_type=[
        jax.ShapeDtypeStruct((NW, 1, N_NODES), jnp.float32),
        jax.ShapeDtypeStruct((NW, 1, N_EDGES), jnp.float32),
    ],
    scratch_types=[
        pltpu.VMEM((INC_PER_W,), jnp.int32),
        pltpu.VMEM((N_NODES,), jnp.float32),
    ],
    compiler_params=_sc_params,
)
def _hist(nidx_hbm, hidx_hbm, dn_out, de_out, idx_v, cnt_v):
    wid = lax.axis_index("s") * NC + lax.axis_index("c")
    ones = jnp.ones((LANES,), jnp.float32)
    zeros = jnp.zeros((LANES,), jnp.float32)

    for src, out in ((nidx_hbm, dn_out), (hidx_hbm, de_out)):
        @pl.loop(0, N_NODES, step=LANES)
        def _zero(i):
            cnt_v[pl.ds(i, LANES)] = zeros

        pltpu.sync_copy(src.at[wid, 0], idx_v)

        @pl.loop(0, INC_PER_W, step=LANES)
        def _accum(i):
            idx = idx_v[pl.ds(i, LANES)]
            plsc.addupdate_scatter(cnt_v, [idx], ones)

        pltpu.sync_copy(cnt_v, out.at[wid, 0])


@functools.partial(
    pl.kernel,
    mesh=_mesh,
    out_type=jax.ShapeDtypeStruct((ACC_N, CH), jnp.float32),
    scratch_types=[
        pltpu.VMEM((1, CHUNK), jnp.int32),
        pltpu.VMEM((1, CHUNK), jnp.int32),
        pltpu.VMEM((1, CHUNK), jnp.int32),
        pltpu.VMEM((1, CHUNK), jnp.int32),
        pltpu.VMEM((CHUNK, CH), jnp.float32),
        pltpu.VMEM((CHUNK, CH), jnp.float32),
        pltpu.VMEM_SHARED((N_NODES, CH), jnp.float32),
        pltpu.VMEM_SHARED((ACC_H, CH), jnp.float32),
        pltpu.SemaphoreType.DMA,
        pltpu.SemaphoreType.DMA,
        pltpu.SemaphoreType.DMA,
        pltpu.SemaphoreType.DMA,
    ],
    compiler_params=_sc_params,
)
def _segpass(table_hbm, gidx_hbm, sidx_hbm, out_hbm,
             gi0, gi1, si0, si1, rows0_v, rows1_v, tab_sh, acc_sh,
             semg0, semg1, semi0, semi1):
    c = lax.axis_index("c")
    s = lax.axis_index("s")
    sbase = s * N_CHUNKS
    lo = c * OWN
    zeros = jnp.zeros((LANES,), jnp.float32)

    def _remap(si):
        for k in range(CHUNK // LANES):
            v = si[0, pl.ds(k * LANES, LANES)]
            local = v - lo
            ok = (local >= 0) & (local < OWN)
            si[0, pl.ds(k * LANES, LANES)] = jnp.where(ok, local, DEAD)

    pltpu.sync_copy(table_hbm.at[pl.ds(s * 624, 624)],
                    tab_sh.at[pl.ds(s * 624, 624)])

    @pl.when(s == NS - 1)
    def _tail():
        pltpu.sync_copy(table_hbm.at[pl.ds(624 * NS, N_NODES - 624 * NS)],
                        tab_sh.at[pl.ds(624 * NS, N_NODES - 624 * NS)])

    @pl.loop(0, CHUNK)
    def _zrow(i):
        @pl.loop(0, CH, step=LANES)
        def _zcol(j):
            rows0_v[i, pl.ds(j, LANES)] = zeros

    @pl.loop(0, ZPT // CHUNK)
    def _zcp(k):
        pltpu.sync_copy(rows0_v, acc_sh.at[pl.ds(s * ZPT + k * CHUNK, CHUNK)])
    _ztail = ZPT % CHUNK
    if _ztail:
        pltpu.sync_copy(
            rows0_v.at[pl.ds(0, _ztail)],
            acc_sh.at[pl.ds(s * ZPT + (ZPT // CHUNK) * CHUNK, _ztail)])

    pltpu.sync_copy(gidx_hbm.at[sbase, 0], gi0.at[0])
    pltpu.sync_copy(sidx_hbm.at[sbase, 0], si0.at[0])
    pltpu.async_copy(gidx_hbm.at[sbase + 1, 0], gi1.at[0], semi1)
    pltpu.async_copy(sidx_hbm.at[sbase + 1, 0], si1.at[0], semi1)

    plsc.subcore_barrier()

    pltpu.async_copy(tab_sh.at[gi0.at[0]], rows0_v, semg0)

    @pl.loop(0, N_CHUNKS, step=2)
    def _chunk(ci):
        pltpu.make_async_copy(tab_sh.at[gi0.at[0]], rows0_v, semg0).wait()
        pltpu.make_async_copy(gidx_hbm.at[sbase, 0], gi1.at[0], semi1).wait()
        pltpu.make_async_copy(sidx_hbm.at[sbase, 0], si1.at[0], semi1).wait()
        pltpu.async_copy(tab_sh.at[gi1.at[0]], rows1_v, semg1)
        _remap(si0)
        pltpu.sync_copy(rows0_v, acc_sh.at[si0.at[0]], add=True)

        @pl.when(ci + 2 < N_CHUNKS)
        def _pf0():
            pltpu.async_copy(gidx_hbm.at[sbase + ci + 2, 0], gi0.at[0], semi0)
            pltpu.async_copy(sidx_hbm.at[sbase + ci + 2, 0], si0.at[0], semi0)

        pltpu.make_async_copy(tab_sh.at[gi1.at[0]], rows1_v, semg1).wait()

        @pl.when(ci + 2 < N_CHUNKS)
        def _g0():
            pltpu.make_async_copy(gidx_hbm.at[sbase, 0], gi0.at[0],
                                  semi0).wait()
            pltpu.make_async_copy(sidx_hbm.at[sbase, 0], si0.at[0],
                                  semi0).wait()
            pltpu.async_copy(tab_sh.at[gi0.at[0]], rows0_v, semg0)

        _remap(si1)
        pltpu.sync_copy(rows1_v, acc_sh.at[si1.at[0]], add=True)

        @pl.when(ci + 3 < N_CHUNKS)
        def _pf1():
            pltpu.async_copy(gidx_hbm.at[sbase + ci + 3, 0], gi1.at[0], semi1)
            pltpu.async_copy(sidx_hbm.at[sbase + ci + 3, 0], si1.at[0], semi1)

    plsc.subcore_barrier()

    pltpu.sync_copy(acc_sh.at[pl.ds(s * DPT, DPT)],
                    out_hbm.at[pl.ds(c * OWN + s * DPT, DPT)])


_BM = 1000


def _scales_body(dnp_ref, dep_ref, dns_ref, dei_ref):
    dn = jnp.sum(dnp_ref[...].T, axis=1, keepdims=True)
    dns_ref[...] = jnp.where(dn > 0, lax.rsqrt(jnp.maximum(dn, 1e-12)), 0.0)
    de = jnp.sum(dep_ref[...].T, axis=1, keepdims=True)
    dei_ref[...] = jnp.where(de > 0, 1.0 / jnp.maximum(de, 1e-12), 0.0)


def _scales(dn_p, de_p):
    return pl.pallas_call(
        _scales_body,
        out_shape=[jax.ShapeDtypeStruct((N_NODES, 1), jnp.float32),
                   jax.ShapeDtypeStruct((N_EDGES, 1), jnp.float32)],
    )(dn_p, de_p)


def _proj_body(x_ref, wt_ref, b_ref, dns_ref, h_ref):
    xw = jnp.dot(x_ref[...], wt_ref[...],
                 preferred_element_type=jnp.float32) + b_ref[...]
    h_ref[...] = xw * dns_ref[...]


def _proj(x, wt, b2, dn_s):
    return pl.pallas_call(
        _proj_body,
        grid=(N_NODES // _BM,),
        in_specs=[
            pl.BlockSpec((_BM, CH), lambda i: (i, 0)),
            pl.BlockSpec((CH, CH), lambda i: (0, 0)),
            pl.BlockSpec((1, CH), lambda i: (0, 0)),
            pl.BlockSpec((_BM, 1), lambda i: (i, 0)),
        ],
        out_specs=pl.BlockSpec((_BM, CH), lambda i: (i, 0)),
        out_shape=jax.ShapeDtypeStruct((N_NODES, CH), jnp.float32),
    )(x, wt, b2, dn_s)


def _combine_body(relu, p_ref, s_ref, o_ref):
    tot = p_ref[...] * s_ref[...]
    o_ref[...] = jnp.maximum(tot, 0.0) if relu else tot


def _combine(p, s, relu):
    return pl.pallas_call(
        functools.partial(_combine_body, relu),
        grid=(N_NODES // _BM,),
        in_specs=[
            pl.BlockSpec((_BM, CH), lambda i: (i, 0)),
            pl.BlockSpec((_BM, 1), lambda i: (i, 0)),
        ],
        out_specs=pl.BlockSpec((_BM, CH), lambda i: (i, 0)),
        out_shape=jax.ShapeDtypeStruct((N_NODES, CH), jnp.float32),
    )(p, s)


def kernel(x, hyperedge_index, W, b):
    nidx = hyperedge_index[0]
    hidx = hyperedge_index[1]
    pad_n = INC_PAD - INC_PER_T

    def _chunk_idx(idx, pad_val):
        idx2 = idx.reshape(NS, INC_PER_T)
        pad = jnp.full((NS, pad_n), pad_val, jnp.int32)
        return jnp.concatenate([idx2, pad], axis=1).reshape(
            NS * N_CHUNKS, 1, CHUNK)

    nidx_g = _chunk_idx(nidx, 0)
    nidx_s = _chunk_idx(nidx, PAD_ROW)
    hidx_g = _chunk_idx(hidx, 0)
    hidx_s = _chunk_idx(hidx, PAD_ROW)
    nidx_w = nidx.reshape(NW, 1, INC_PER_W)
    hidx_w = hidx.reshape(NW, 1, INC_PER_W)
    wt = W.T
    b2 = b.reshape(1, CH)

    dn_p, de_p = _hist(nidx_w, hidx_w)
    dn_s, de_i = _scales(dn_p.reshape(NW, N_NODES), de_p.reshape(NW, N_EDGES))
    h = _proj(x, wt, b2, dn_s)
    e_p = _segpass(h, nidx_g, hidx_s)
    e = _combine(e_p, de_i, relu=False)
    y_p = _segpass(e, hidx_g, nidx_s)
    y = _combine(y_p, dn_s, relu=True)
    return y


# speedup vs baseline: 1.4976x; 1.4976x over previous
import functools

import jax
import jax.numpy as jnp
from jax import lax
from jax.experimental import pallas as pl
from jax.experimental.pallas import tpu as pltpu
from jax.experimental.pallas import tpu_sc as plsc

N_NODES = 10000
N_EDGES = 10000
N_INC = 320000
CH = 128

NC = 2
NS = 16
NW = NC * NS
LANES = 16

INC_PER_W = N_INC // NW
CHUNK = 80
N_CHUNKS = INC_PER_W // CHUNK
ACC_N = 10240
ROWS_PER_TILE = ACC_N // NS
ZB_ROWS = 128

_mesh = plsc.VectorSubcoreMesh(core_axis_name="c", subcore_axis_name="s")
_sc_params = pltpu.CompilerParams(needs_layout_passes=False)


@functools.partial(
    pl.kernel,
    mesh=_mesh,
    out_type=[
        jax.ShapeDtypeStruct((NW, 1, N_NODES), jnp.float32),
        jax.ShapeDtypeStruct((NW, 1, N_EDGES), jnp.float32),
    ],
    scratch_types=[
        pltpu.VMEM((INC_PER_W,), jnp.int32),
        pltpu.VMEM((N_NODES,), jnp.float32),
    ],
    compiler_params=_sc_params,
)
def _hist(nidx_hbm, hidx_hbm, dn_out, de_out, idx_v, cnt_v):
    wid = lax.axis_index("s") * NC + lax.axis_index("c")
    ones = jnp.ones((LANES,), jnp.float32)
    zeros = jnp.zeros((LANES,), jnp.float32)

    for src, out in ((nidx_hbm, dn_out), (hidx_hbm, de_out)):
        @pl.loop(0, N_NODES, step=LANES)
        def _zero(i):
            cnt_v[pl.ds(i, LANES)] = zeros

        pltpu.sync_copy(src.at[wid, 0], idx_v)

        @pl.loop(0, INC_PER_W, step=LANES)
        def _accum(i):
            idx = idx_v[pl.ds(i, LANES)]
            plsc.addupdate_scatter(cnt_v, [idx], ones)

        pltpu.sync_copy(cnt_v, out.at[wid, 0])


@functools.partial(
    pl.kernel,
    mesh=_mesh,
    out_type=jax.ShapeDtypeStruct((NC, ACC_N, CH), jnp.float32),
    scratch_types=[
        pltpu.VMEM((1, CHUNK), jnp.int32),
        pltpu.VMEM((1, CHUNK), jnp.int32),
        pltpu.VMEM((CHUNK, CH), jnp.float32),
        pltpu.VMEM((ZB_ROWS, CH), jnp.float32),
        pltpu.VMEM_SHARED((ACC_N, CH), jnp.float32),
        pltpu.SemaphoreType.DMA,
    ],
    compiler_params=_sc_params,
)
def _segpass(table_hbm, gidx_hbm, sidx_hbm, out_hbm,
             gi_v, si_v, rows_v, zb_v, acc_sh, sem):
    c = lax.axis_index("c")
    s = lax.axis_index("s")
    wid = s * NC + c
    zeros = jnp.zeros((LANES,), jnp.float32)

    @pl.loop(0, ZB_ROWS)
    def _zrow(i):
        @pl.loop(0, CH, step=LANES)
        def _zcol(j):
            zb_v[i, pl.ds(j, LANES)] = zeros

    @pl.loop(0, ROWS_PER_TILE // ZB_ROWS)
    def _zcp(k):
        pltpu.sync_copy(zb_v, acc_sh.at[pl.ds(s * ROWS_PER_TILE + k * ZB_ROWS,
                                              ZB_ROWS)])

    plsc.subcore_barrier()

    @pl.loop(0, N_CHUNKS)
    def _chunk(ci):
        blk = wid * N_CHUNKS + ci
        pltpu.sync_copy(gidx_hbm.at[blk, 0], gi_v.at[0])
        pltpu.sync_copy(sidx_hbm.at[blk, 0], si_v.at[0])
        pltpu.async_copy(table_hbm.at[gi_v.at[0]], rows_v, sem).wait()
        pltpu.sync_copy(rows_v, acc_sh.at[si_v.at[0]], add=True)

    plsc.subcore_barrier()

    pltpu.sync_copy(acc_sh.at[pl.ds(s * ROWS_PER_TILE, ROWS_PER_TILE)],
                    out_hbm.at[c, pl.ds(s * ROWS_PER_TILE, ROWS_PER_TILE)])


_BM = 1000


def _scales_body(dnp_ref, dep_ref, dns_ref, dei_ref):
    dn = jnp.sum(dnp_ref[...].T, axis=1, keepdims=True)
    dns_ref[...] = jnp.where(dn > 0, lax.rsqrt(jnp.maximum(dn, 1e-12)), 0.0)
    de = jnp.sum(dep_ref[...].T, axis=1, keepdims=True)
    dei_ref[...] = jnp.where(de > 0, 1.0 / jnp.maximum(de, 1e-12), 0.0)


def _scales(dn_p, de_p):
    return pl.pallas_call(
        _scales_body,
        out_shape=[jax.ShapeDtypeStruct((N_NODES, 1), jnp.float32),
                   jax.ShapeDtypeStruct((N_EDGES, 1), jnp.float32)],
    )(dn_p, de_p)


def _proj_body(x_ref, wt_ref, b_ref, dns_ref, h_ref):
    xw = jnp.dot(x_ref[...], wt_ref[...],
                 preferred_element_type=jnp.float32) + b_ref[...]
    h_ref[...] = xw * dns_ref[...]


def _proj(x, wt, b2, dn_s):
    return pl.pallas_call(
        _proj_body,
        grid=(N_NODES // _BM,),
        in_specs=[
            pl.BlockSpec((_BM, CH), lambda i: (i, 0)),
            pl.BlockSpec((CH, CH), lambda i: (0, 0)),
            pl.BlockSpec((1, CH), lambda i: (0, 0)),
            pl.BlockSpec((_BM, 1), lambda i: (i, 0)),
        ],
        out_specs=pl.BlockSpec((_BM, CH), lambda i: (i, 0)),
        out_shape=jax.ShapeDtypeStruct((N_NODES, CH), jnp.float32),
    )(x, wt, b2, dn_s)


def _combine_body(relu, p_ref, s_ref, o_ref):
    tot = (p_ref[0] + p_ref[1]) * s_ref[...]
    o_ref[...] = jnp.maximum(tot, 0.0) if relu else tot


def _combine(p, s, relu):
    return pl.pallas_call(
        functools.partial(_combine_body, relu),
        grid=(N_NODES // _BM,),
        in_specs=[
            pl.BlockSpec((NC, _BM, CH), lambda i: (0, i, 0)),
            pl.BlockSpec((_BM, 1), lambda i: (i, 0)),
        ],
        out_specs=pl.BlockSpec((_BM, CH), lambda i: (i, 0)),
        out_shape=jax.ShapeDtypeStruct((N_NODES, CH), jnp.float32),
    )(p, s)


def kernel(x, hyperedge_index, W, b):
    nidx = hyperedge_index[0]
    hidx = hyperedge_index[1]
    nidx_c = nidx.reshape(NW * N_CHUNKS, 1, CHUNK)
    hidx_c = hidx.reshape(NW * N_CHUNKS, 1, CHUNK)
    nidx_w = nidx.reshape(NW, 1, INC_PER_W)
    hidx_w = hidx.reshape(NW, 1, INC_PER_W)
    wt = W.T
    b2 = b.reshape(1, CH)

    dn_p, de_p = _hist(nidx_w, hidx_w)
    dn_s, de_i = _scales(dn_p.reshape(NW, N_NODES), de_p.reshape(NW, N_EDGES))
    h = _proj(x, wt, b2, dn_s)
    e_p = _segpass(h, nidx_c, hidx_c)
    e = _combine(e_p, de_i, relu=False)
    y_p = _segpass(e, hidx_c, nidx_c)
    y = _combine(y_p, dn_s, relu=True)
    return y


# speedup vs baseline: 2.1640x; 1.4450x over previous
import functools

import jax
import jax.numpy as jnp
from jax import lax
from jax.experimental import pallas as pl
from jax.experimental.pallas import tpu as pltpu
from jax.experimental.pallas import tpu_sc as plsc

N_NODES = 10000
N_EDGES = 10000
N_INC = 320000
CH = 128

NC = 2
NS = 16
NW = NC * NS
LANES = 16

INC_PER_W = N_INC // NW
CHUNK = 80
N_CHUNKS = INC_PER_W // CHUNK
ACC_N = 10240
ROWS_PER_TILE = ACC_N // NS
ZB_ROWS = 128

_mesh = plsc.VectorSubcoreMesh(core_axis_name="c", subcore_axis_name="s")
_sc_params = pltpu.CompilerParams(needs_layout_passes=False)


@functools.partial(
    pl.kernel,
    mesh=_mesh,
    out_type=[
        jax.ShapeDtypeStruct((NW, 1, N_NODES), jnp.float32),
        jax.ShapeDtypeStruct((NW, 1, N_EDGES), jnp.float32),
    ],
    scratch_types=[
        pltpu.VMEM((INC_PER_W,), jnp.int32),
        pltpu.VMEM((N_NODES,), jnp.float32),
    ],
    compiler_params=_sc_params,
)
def _hist(nidx_hbm, hidx_hbm, dn_out, de_out, idx_v, cnt_v):
    wid = lax.axis_index("s") * NC + lax.axis_index("c")
    ones = jnp.ones((LANES,), jnp.float32)
    zeros = jnp.zeros((LANES,), jnp.float32)

    for src, out in ((nidx_hbm, dn_out), (hidx_hbm, de_out)):
        @pl.loop(0, N_NODES, step=LANES)
        def _zero(i):
            cnt_v[pl.ds(i, LANES)] = zeros

        pltpu.sync_copy(src.at[wid, 0], idx_v)

        @pl.loop(0, INC_PER_W, step=LANES)
        def _accum(i):
            idx = idx_v[pl.ds(i, LANES)]
            plsc.addupdate_scatter(cnt_v, [idx], ones)

        pltpu.sync_copy(cnt_v, out.at[wid, 0])


@functools.partial(
    pl.kernel,
    mesh=_mesh,
    out_type=jax.ShapeDtypeStruct((NC, ACC_N, CH), jnp.float32),
    scratch_types=[
        pltpu.VMEM((1, CHUNK), jnp.int32),
        pltpu.VMEM((1, CHUNK), jnp.int32),
        pltpu.VMEM((1, CHUNK), jnp.int32),
        pltpu.VMEM((1, CHUNK), jnp.int32),
        pltpu.VMEM((CHUNK, CH), jnp.float32),
        pltpu.VMEM((ZB_ROWS, CH), jnp.float32),
        pltpu.VMEM_SHARED((ACC_N, CH), jnp.float32),
        pltpu.SemaphoreType.DMA,
        pltpu.SemaphoreType.DMA,
        pltpu.SemaphoreType.DMA,
    ],
    compiler_params=_sc_params,
)
def _segpass(table_hbm, gidx_hbm, sidx_hbm, out_hbm,
             gi0, si0, gi1, si1, rows_v, zb_v, acc_sh, sem, semi0, semi1):
    c = lax.axis_index("c")
    s = lax.axis_index("s")
    wid = s * NC + c
    base = wid * N_CHUNKS
    zeros = jnp.zeros((LANES,), jnp.float32)

    @pl.loop(0, ZB_ROWS)
    def _zrow(i):
        @pl.loop(0, CH, step=LANES)
        def _zcol(j):
            zb_v[i, pl.ds(j, LANES)] = zeros

    @pl.loop(0, ROWS_PER_TILE // ZB_ROWS)
    def _zcp(k):
        pltpu.sync_copy(zb_v, acc_sh.at[pl.ds(s * ROWS_PER_TILE + k * ZB_ROWS,
                                              ZB_ROWS)])

    pltpu.async_copy(gidx_hbm.at[base, 0], gi0.at[0], semi0)
    pltpu.async_copy(sidx_hbm.at[base, 0], si0.at[0], semi0)
    pltpu.async_copy(gidx_hbm.at[base + 1, 0], gi1.at[0], semi1)
    pltpu.async_copy(sidx_hbm.at[base + 1, 0], si1.at[0], semi1)

    plsc.subcore_barrier()

    @pl.loop(0, N_CHUNKS, step=2)
    def _chunk(ci):
        pltpu.make_async_copy(gidx_hbm.at[base, 0], gi0.at[0], semi0).wait()
        pltpu.make_async_copy(sidx_hbm.at[base, 0], si0.at[0], semi0).wait()
        pltpu.async_copy(table_hbm.at[gi0.at[0]], rows_v, sem).wait()
        pltpu.sync_copy(rows_v, acc_sh.at[si0.at[0]], add=True)

        @pl.when(ci + 2 < N_CHUNKS)
        def _pf0():
            pltpu.async_copy(gidx_hbm.at[base + ci + 2, 0], gi0.at[0], semi0)
            pltpu.async_copy(sidx_hbm.at[base + ci + 2, 0], si0.at[0], semi0)

        @pl.when(ci + 1 < N_CHUNKS)
        def _odd():
            pltpu.make_async_copy(gidx_hbm.at[base, 0], gi1.at[0],
                                  semi1).wait()
            pltpu.make_async_copy(sidx_hbm.at[base, 0], si1.at[0],
                                  semi1).wait()
            pltpu.async_copy(table_hbm.at[gi1.at[0]], rows_v, sem).wait()
            pltpu.sync_copy(rows_v, acc_sh.at[si1.at[0]], add=True)

            @pl.when(ci + 3 < N_CHUNKS)
            def _pf1():
                pltpu.async_copy(gidx_hbm.at[base + ci + 3, 0], gi1.at[0],
                                 semi1)
                pltpu.async_copy(sidx_hbm.at[base + ci + 3, 0], si1.at[0],
                                 semi1)

    plsc.subcore_barrier()

    pltpu.sync_copy(acc_sh.at[pl.ds(s * ROWS_PER_TILE, ROWS_PER_TILE)],
                    out_hbm.at[c, pl.ds(s * ROWS_PER_TILE, ROWS_PER_TILE)])


_BM = 1000


def _scales_body(dnp_ref, dep_ref, dns_ref, dei_ref):
    dn = jnp.sum(dnp_ref[...].T, axis=1, keepdims=True)
    dns_ref[...] = jnp.where(dn > 0, lax.rsqrt(jnp.maximum(dn, 1e-12)), 0.0)
    de = jnp.sum(dep_ref[...].T, axis=1, keepdims=True)
    dei_ref[...] = jnp.where(de > 0, 1.0 / jnp.maximum(de, 1e-12), 0.0)


def _scales(dn_p, de_p):
    return pl.pallas_call(
        _scales_body,
        out_shape=[jax.ShapeDtypeStruct((N_NODES, 1), jnp.float32),
                   jax.ShapeDtypeStruct((N_EDGES, 1), jnp.float32)],
    )(dn_p, de_p)


def _proj_body(x_ref, wt_ref, b_ref, dns_ref, h_ref):
    xw = jnp.dot(x_ref[...], wt_ref[...],
                 preferred_element_type=jnp.float32) + b_ref[...]
    h_ref[...] = xw * dns_ref[...]


def _proj(x, wt, b2, dn_s):
    return pl.pallas_call(
        _proj_body,
        grid=(N_NODES // _BM,),
        in_specs=[
            pl.BlockSpec((_BM, CH), lambda i: (i, 0)),
            pl.BlockSpec((CH, CH), lambda i: (0, 0)),
            pl.BlockSpec((1, CH), lambda i: (0, 0)),
            pl.BlockSpec((_BM, 1), lambda i: (i, 0)),
        ],
        out_specs=pl.BlockSpec((_BM, CH), lambda i: (i, 0)),
        out_shape=jax.ShapeDtypeStruct((N_NODES, CH), jnp.float32),
    )(x, wt, b2, dn_s)


def _combine_body(relu, p_ref, s_ref, o_ref):
    tot = (p_ref[0] + p_ref[1]) * s_ref[...]
    o_ref[...] = jnp.maximum(tot, 0.0) if relu else tot


def _combine(p, s, relu):
    return pl.pallas_call(
        functools.partial(_combine_body, relu),
        grid=(N_NODES // _BM,),
        in_specs=[
            pl.BlockSpec((NC, _BM, CH), lambda i: (0, i, 0)),
            pl.BlockSpec((_BM, 1), lambda i: (i, 0)),
        ],
        out_specs=pl.BlockSpec((_BM, CH), lambda i: (i, 0)),
        out_shape=jax.ShapeDtypeStruct((N_NODES, CH), jnp.float32),
    )(p, s)


def kernel(x, hyperedge_index, W, b):
    nidx = hyperedge_index[0]
    hidx = hyperedge_index[1]
    nidx_c = nidx.reshape(NW * N_CHUNKS, 1, CHUNK)
    hidx_c = hidx.reshape(NW * N_CHUNKS, 1, CHUNK)
    nidx_w = nidx.reshape(NW, 1, INC_PER_W)
    hidx_w = hidx.reshape(NW, 1, INC_PER_W)
    wt = W.T
    b2 = b.reshape(1, CH)

    dn_p, de_p = _hist(nidx_w, hidx_w)
    dn_s, de_i = _scales(dn_p.reshape(NW, N_NODES), de_p.reshape(NW, N_EDGES))
    h = _proj(x, wt, b2, dn_s)
    e_p = _segpass(h, nidx_c, hidx_c)
    e = _combine(e_p, de_i, relu=False)
    y_p = _segpass(e, hidx_c, nidx_c)
    y = _combine(y_p, dn_s, relu=True)
    return y


# speedup vs baseline: 2.7002x; 1.2478x over previous
import functools

import jax
import jax.numpy as jnp
from jax import lax
from jax.experimental import pallas as pl
from jax.experimental.pallas import tpu as pltpu
from jax.experimental.pallas import tpu_sc as plsc

N_NODES = 10000
N_EDGES = 10000
N_INC = 320000
CH = 128

NC = 2
NS = 16
NW = NC * NS
LANES = 16

INC_PER_W = N_INC // NW
CHUNK = 80
N_CHUNKS = INC_PER_W // CHUNK
ACC_N = 10240
ROWS_PER_TILE = ACC_N // NS
ZB_ROWS = 128

_mesh = plsc.VectorSubcoreMesh(core_axis_name="c", subcore_axis_name="s")
_sc_params = pltpu.CompilerParams(needs_layout_passes=False)


@functools.partial(
    pl.kernel,
    mesh=_mesh,
    out_type=[
        jax.ShapeDtypeStruct((NW, 1, N_NODES), jnp.float32),
        jax.ShapeDtypeStruct((NW, 1, N_EDGES), jnp.float32),
    ],
    scratch_types=[
        pltpu.VMEM((INC_PER_W,), jnp.int32),
        pltpu.VMEM((N_NODES,), jnp.float32),
    ],
    compiler_params=_sc_params,
)
def _hist(nidx_hbm, hidx_hbm, dn_out, de_out, idx_v, cnt_v):
    wid = lax.axis_index("s") * NC + lax.axis_index("c")
    ones = jnp.ones((LANES,), jnp.float32)
    zeros = jnp.zeros((LANES,), jnp.float32)

    for src, out in ((nidx_hbm, dn_out), (hidx_hbm, de_out)):
        @pl.loop(0, N_NODES, step=LANES)
        def _zero(i):
            cnt_v[pl.ds(i, LANES)] = zeros

        pltpu.sync_copy(src.at[wid, 0], idx_v)

        @pl.loop(0, INC_PER_W, step=LANES)
        def _accum(i):
            idx = idx_v[pl.ds(i, LANES)]
            plsc.addupdate_scatter(cnt_v, [idx], ones)

        pltpu.sync_copy(cnt_v, out.at[wid, 0])


@functools.partial(
    pl.kernel,
    mesh=_mesh,
    out_type=jax.ShapeDtypeStruct((NC, ACC_N, CH), jnp.float32),
    scratch_types=[
        pltpu.VMEM((1, CHUNK), jnp.int32),
        pltpu.VMEM((1, CHUNK), jnp.int32),
        pltpu.VMEM((1, CHUNK), jnp.int32),
        pltpu.VMEM((1, CHUNK), jnp.int32),
        pltpu.VMEM((CHUNK, CH), jnp.float32),
        pltpu.VMEM((CHUNK, CH), jnp.float32),
        pltpu.VMEM((ZB_ROWS, CH), jnp.float32),
        pltpu.VMEM_SHARED((ACC_N, CH), jnp.float32),
        pltpu.SemaphoreType.DMA,
        pltpu.SemaphoreType.DMA,
        pltpu.SemaphoreType.DMA,
        pltpu.SemaphoreType.DMA,
    ],
    compiler_params=_sc_params,
)
def _segpass(table_hbm, gidx_hbm, sidx_hbm, out_hbm,
             gi0, si0, gi1, si1, rows0_v, rows1_v, zb_v, acc_sh,
             sem0, sem1, semi0, semi1):
    c = lax.axis_index("c")
    s = lax.axis_index("s")
    wid = s * NC + c
    base = wid * N_CHUNKS
    zeros = jnp.zeros((LANES,), jnp.float32)

    @pl.loop(0, ZB_ROWS)
    def _zrow(i):
        @pl.loop(0, CH, step=LANES)
        def _zcol(j):
            zb_v[i, pl.ds(j, LANES)] = zeros

    @pl.loop(0, ROWS_PER_TILE // ZB_ROWS)
    def _zcp(k):
        pltpu.sync_copy(zb_v, acc_sh.at[pl.ds(s * ROWS_PER_TILE + k * ZB_ROWS,
                                              ZB_ROWS)])

    pltpu.async_copy(gidx_hbm.at[base, 0], gi0.at[0], semi0)
    pltpu.async_copy(sidx_hbm.at[base, 0], si0.at[0], semi0)
    pltpu.async_copy(gidx_hbm.at[base + 1, 0], gi1.at[0], semi1)
    pltpu.async_copy(sidx_hbm.at[base + 1, 0], si1.at[0], semi1)

    plsc.subcore_barrier()

    pltpu.make_async_copy(gidx_hbm.at[base, 0], gi0.at[0], semi0).wait()
    pltpu.make_async_copy(sidx_hbm.at[base, 0], si0.at[0], semi0).wait()
    pltpu.async_copy(table_hbm.at[gi0.at[0]], rows0_v, sem0)

    @pl.loop(0, N_CHUNKS, step=2)
    def _chunk(ci):
        pltpu.make_async_copy(table_hbm.at[gi0.at[0]], rows0_v, sem0).wait()

        @pl.when(ci + 1 < N_CHUNKS)
        def _g1():
            pltpu.make_async_copy(gidx_hbm.at[base, 0], gi1.at[0],
                                  semi1).wait()
            pltpu.make_async_copy(sidx_hbm.at[base, 0], si1.at[0],
                                  semi1).wait()
            pltpu.async_copy(table_hbm.at[gi1.at[0]], rows1_v, sem1)

        pltpu.sync_copy(rows0_v, acc_sh.at[si0.at[0]], add=True)

        @pl.when(ci + 2 < N_CHUNKS)
        def _pf0():
            pltpu.async_copy(gidx_hbm.at[base + ci + 2, 0], gi0.at[0], semi0)
            pltpu.async_copy(sidx_hbm.at[base + ci + 2, 0], si0.at[0], semi0)

        @pl.when(ci + 1 < N_CHUNKS)
        def _odd():
            pltpu.make_async_copy(table_hbm.at[gi1.at[0]], rows1_v,
                                  sem1).wait()

            @pl.when(ci + 2 < N_CHUNKS)
            def _g2():
                pltpu.make_async_copy(gidx_hbm.at[base, 0], gi0.at[0],
                                      semi0).wait()
                pltpu.make_async_copy(sidx_hbm.at[base, 0], si0.at[0],
                                      semi0).wait()
                pltpu.async_copy(table_hbm.at[gi0.at[0]], rows0_v, sem0)

            pltpu.sync_copy(rows1_v, acc_sh.at[si1.at[0]], add=True)

            @pl.when(ci + 3 < N_CHUNKS)
            def _pf1():
                pltpu.async_copy(gidx_hbm.at[base + ci + 3, 0], gi1.at[0],
                                 semi1)
                pltpu.async_copy(sidx_hbm.at[base + ci + 3, 0], si1.at[0],
                                 semi1)

    plsc.subcore_barrier()

    pltpu.sync_copy(acc_sh.at[pl.ds(s * ROWS_PER_TILE, ROWS_PER_TILE)],
                    out_hbm.at[c, pl.ds(s * ROWS_PER_TILE, ROWS_PER_TILE)])


_BM = 1000


def _scales_body(dnp_ref, dep_ref, dns_ref, dei_ref):
    dn = jnp.sum(dnp_ref[...].T, axis=1, keepdims=True)
    dns_ref[...] = jnp.where(dn > 0, lax.rsqrt(jnp.maximum(dn, 1e-12)), 0.0)
    de = jnp.sum(dep_ref[...].T, axis=1, keepdims=True)
    dei_ref[...] = jnp.where(de > 0, 1.0 / jnp.maximum(de, 1e-12), 0.0)


def _scales(dn_p, de_p):
    return pl.pallas_call(
        _scales_body,
        out_shape=[jax.ShapeDtypeStruct((N_NODES, 1), jnp.float32),
                   jax.ShapeDtypeStruct((N_EDGES, 1), jnp.float32)],
    )(dn_p, de_p)


def _proj_body(x_ref, wt_ref, b_ref, dns_ref, h_ref):
    xw = jnp.dot(x_ref[...], wt_ref[...],
                 preferred_element_type=jnp.float32) + b_ref[...]
    h_ref[...] = xw * dns_ref[...]


def _proj(x, wt, b2, dn_s):
    return pl.pallas_call(
        _proj_body,
        grid=(N_NODES // _BM,),
        in_specs=[
            pl.BlockSpec((_BM, CH), lambda i: (i, 0)),
            pl.BlockSpec((CH, CH), lambda i: (0, 0)),
            pl.BlockSpec((1, CH), lambda i: (0, 0)),
            pl.BlockSpec((_BM, 1), lambda i: (i, 0)),
        ],
        out_specs=pl.BlockSpec((_BM, CH), lambda i: (i, 0)),
        out_shape=jax.ShapeDtypeStruct((N_NODES, CH), jnp.float32),
    )(x, wt, b2, dn_s)


def _combine_body(relu, p_ref, s_ref, o_ref):
    tot = (p_ref[0] + p_ref[1]) * s_ref[...]
    o_ref[...] = jnp.maximum(tot, 0.0) if relu else tot


def _combine(p, s, relu):
    return pl.pallas_call(
        functools.partial(_combine_body, relu),
        grid=(N_NODES // _BM,),
        in_specs=[
            pl.BlockSpec((NC, _BM, CH), lambda i: (0, i, 0)),
            pl.BlockSpec((_BM, 1), lambda i: (i, 0)),
        ],
        out_specs=pl.BlockSpec((_BM, CH), lambda i: (i, 0)),
        out_shape=jax.ShapeDtypeStruct((N_NODES, CH), jnp.float32),
    )(p, s)


def kernel(x, hyperedge_index, W, b):
    nidx = hyperedge_index[0]
    hidx = hyperedge_index[1]
    nidx_c = nidx.reshape(NW * N_CHUNKS, 1, CHUNK)
    hidx_c = hidx.reshape(NW * N_CHUNKS, 1, CHUNK)
    nidx_w = nidx.reshape(NW, 1, INC_PER_W)
    hidx_w = hidx.reshape(NW, 1, INC_PER_W)
    wt = W.T
    b2 = b.reshape(1, CH)

    dn_p, de_p = _hist(nidx_w, hidx_w)
    dn_s, de_i = _scales(dn_p.reshape(NW, N_NODES), de_p.reshape(NW, N_EDGES))
    h = _proj(x, wt, b2, dn_s)
    e_p = _segpass(h, nidx_c, hidx_c)
    e = _combine(e_p, de_i, relu=False)
    y_p = _segpass(e, hidx_c, nidx_c)
    y = _combine(y_p, dn_s, relu=True)
    return y
